# Initial kernel scaffold; baseline (speedup 1.0000x reference)
import jax
import jax.numpy as jnp
from jax.experimental import pallas as pl


def kernel(X, m, center, weight):
    raise NotImplementedError("write your pallas kernel here")


# speedup vs baseline: 2.2967x; 2.2967x over previous
import jax
import jax.numpy as jnp
from jax import lax
from jax.experimental import pallas as pl

_M = 8
_K = 1024
_D = 1024
_Dp = _D // _M
_B = 4096
_BLK = 512


def _rownorm_s8tree(sq):
    v = sq.reshape(sq.shape[:-1] + (16, 8))
    acc = v[..., 0, :]
    for i in range(1, 16):
        acc = acc + v[..., i, :]
    t1 = acc[..., 0:4] + acc[..., 4:8]
    t2 = t1[..., 0:2] + t1[..., 2:4]
    return t2[..., 0] + t2[..., 1]


def _vq_body(x_ref, c_ref, csq_ref, xsq_ref, out_ref, lbl_ref):
    labels = []
    for m in range(_M):
        xm = x_ref[:, m * _Dp:(m + 1) * _Dp]
        cm = c_ref[m]
        s = lax.dot_general(cm, xm, (((1,), (1,)), ((), ())),
                            preferred_element_type=jnp.float32)
        dist = (csq_ref[m][:, None] - 2.0 * s) + xsq_ref[m][None, :]
        val = -jnp.sqrt(dist)
        colmax = jnp.max(val, axis=0, keepdims=True)
        iota = lax.broadcasted_iota(jnp.int32, (_K, _BLK), 0)
        lbl = jnp.min(jnp.where(val == colmax, iota, _K), axis=0)
        labels.append(lbl)
        onehot = (iota == lbl[None, :]).astype(jnp.float32)
        xp = lax.dot_general(onehot, cm, (((0,), (0,)), ((), ())),
                             preferred_element_type=jnp.float32)
        out_ref[:, m * _Dp:(m + 1) * _Dp] = xp
    lbl_ref[...] = jnp.stack(labels, axis=0)


def kernel(X, m, center, weight):
    Xs = lax.stop_gradient(X)
    xsq = _rownorm_s8tree((Xs * Xs).reshape(_B, _M, _Dp)).T
    csq = _rownorm_s8tree(center * center)

    grid = (_B // _BLK,)
    out, lbl = pl.pallas_call(
        _vq_body,
        grid=grid,
        in_specs=[
            pl.BlockSpec((_BLK, _D), lambda i: (i, 0)),
            pl.BlockSpec((_M, _K, _Dp), lambda i: (0, 0, 0)),
            pl.BlockSpec((_M, _K), lambda i: (0, 0)),
            pl.BlockSpec((_M, _BLK), lambda i: (0, i)),
        ],
        out_specs=[
            pl.BlockSpec((_BLK, _D), lambda i: (i, 0)),
            pl.BlockSpec((_M, _BLK), lambda i: (0, i)),
        ],
        out_shape=[
            jax.ShapeDtypeStruct((_B, _D), jnp.float32),
            jax.ShapeDtypeStruct((_M, _B), jnp.int32),
        ],
    )(X, center, csq, xsq)

    X_p = out.reshape(_B, _M, _Dp)
    label = lbl.T.reshape(_B, _M, 1)
    return (X_p, out, center, label, weight)


# speedup vs baseline: 2.2975x; 1.0003x over previous
import jax
import jax.numpy as jnp
from jax import lax
from jax.experimental import pallas as pl
from jax.experimental.pallas import tpu as pltpu

_M = 8
_K = 1024
_D = 1024
_Dp = _D // _M
_B = 4096
_BLK = 512


def _rownorm_s8tree(sq):
    v = sq.reshape(sq.shape[:-1] + (16, 8))
    acc = v[..., 0, :]
    for i in range(1, 16):
        acc = acc + v[..., i, :]
    t1 = acc[..., 0:4] + acc[..., 4:8]
    t2 = t1[..., 0:2] + t1[..., 2:4]
    return t2[..., 0] + t2[..., 1]


def _vq_body(x_ref, c_ref, csq_ref, xsq_ref, out_ref, lbl_ref):
    labels = []
    for m in range(_M):
        xm = x_ref[:, m * _Dp:(m + 1) * _Dp]
        cm = c_ref[m]
        s = lax.dot_general(cm, xm, (((1,), (1,)), ((), ())),
                            preferred_element_type=jnp.float32)
        dist = (csq_ref[m][:, None] - 2.0 * s) + xsq_ref[m][None, :]
        val = -jnp.sqrt(dist)
        colmax = jnp.max(val, axis=0, keepdims=True)
        iota = lax.broadcasted_iota(jnp.int32, (_K, _BLK), 0)
        lbl = jnp.min(jnp.where(val == colmax, iota, _K), axis=0)
        labels.append(lbl)
        onehot = (iota == lbl[None, :]).astype(jnp.float32)
        xp = lax.dot_general(onehot, cm, (((0,), (0,)), ((), ())),
                             preferred_element_type=jnp.float32)
        out_ref[:, m * _Dp:(m + 1) * _Dp] = xp
    lbl_ref[...] = jnp.stack(labels, axis=0)


def kernel(X, m, center, weight):
    Xs = lax.stop_gradient(X)
    xsq = _rownorm_s8tree((Xs * Xs).reshape(_B, _M, _Dp)).T
    csq = _rownorm_s8tree(center * center)

    grid = (_B // _BLK,)
    out, lbl = pl.pallas_call(
        _vq_body,
        grid=grid,
        in_specs=[
            pl.BlockSpec((_BLK, _D), lambda i: (i, 0)),
            pl.BlockSpec((_M, _K, _Dp), lambda i: (0, 0, 0)),
            pl.BlockSpec((_M, _K), lambda i: (0, 0)),
            pl.BlockSpec((_M, _BLK), lambda i: (0, i)),
        ],
        out_specs=[
            pl.BlockSpec((_BLK, _D), lambda i: (i, 0)),
            pl.BlockSpec((_M, _BLK), lambda i: (0, i)),
        ],
        out_shape=[
            jax.ShapeDtypeStruct((_B, _D), jnp.float32),
            jax.ShapeDtypeStruct((_M, _B), jnp.int32),
        ],
        compiler_params=pltpu.CompilerParams(
            dimension_semantics=("parallel",)),
    )(X, center, csq, xsq)

    X_p = out.reshape(_B, _M, _Dp)
    label = lbl.T.reshape(_B, _M, 1)
    return (X_p, out, center, label, weight)


# speedup vs baseline: 2.3130x; 1.0068x over previous
import jax
import jax.numpy as jnp
from jax import lax
from jax.experimental import pallas as pl
from jax.experimental.pallas import tpu as pltpu

_M = 8
_K = 1024
_D = 1024
_Dp = _D // _M
_B = 4096
_BLK = 512


def _rownorm_s8tree(sq):
    v = sq.reshape(sq.shape[:-1] + (16, 8))
    acc = v[..., 0, :]
    for i in range(1, 16):
        acc = acc + v[..., i, :]
    t1 = acc[..., 0:4] + acc[..., 4:8]
    t2 = t1[..., 0:2] + t1[..., 2:4]
    return t2[..., 0] + t2[..., 1]


def _vq_body(x_ref, c2_ref, csq_ref, xsq_ref, out_ref, lbl_ref):
    labels = []
    for m in range(_M):
        xm = x_ref[:, m * _Dp:(m + 1) * _Dp]
        c2m = c2_ref[m]
        s2 = lax.dot_general(c2m, xm, (((1,), (1,)), ((), ())),
                             preferred_element_type=jnp.float32)
        dist = (csq_ref[m][:, None] - s2) + xsq_ref[m][None, :]
        sq = jnp.sqrt(dist)
        sqb = lax.bitcast_convert_type(sq, jnp.int32)
        colmin = jnp.min(sqb, axis=0, keepdims=True)
        iota = lax.broadcasted_iota(jnp.int32, (_K, _BLK), 0)
        lbl = jnp.min(jnp.where(sqb == colmin, iota, _K), axis=0)
        labels.append(lbl)
        onehot = (iota == lbl[None, :]).astype(jnp.float32)
        xp = 0.5 * lax.dot_general(onehot, c2m, (((0,), (0,)), ((), ())),
                                   preferred_element_type=jnp.float32)
        out_ref[:, m * _Dp:(m + 1) * _Dp] = xp
    lbl_ref[...] = jnp.stack(labels, axis=0)


def kernel(X, m, center, weight):
    Xs = lax.stop_gradient(X)
    xsq = _rownorm_s8tree((Xs * Xs).reshape(_B, _M, _Dp)).T
    csq = _rownorm_s8tree(center * center)
    center2 = center + center

    grid = (_B // _BLK,)
    out, lbl = pl.pallas_call(
        _vq_body,
        grid=grid,
        in_specs=[
            pl.BlockSpec((_BLK, _D), lambda i: (i, 0)),
            pl.BlockSpec((_M, _K, _Dp), lambda i: (0, 0, 0)),
            pl.BlockSpec((_M, _K), lambda i: (0, 0)),
            pl.BlockSpec((_M, _BLK), lambda i: (0, i)),
        ],
        out_specs=[
            pl.BlockSpec((_BLK, _D), lambda i: (i, 0)),
            pl.BlockSpec((_M, _BLK), lambda i: (0, i)),
        ],
        out_shape=[
            jax.ShapeDtypeStruct((_B, _D), jnp.float32),
            jax.ShapeDtypeStruct((_M, _B), jnp.int32),
        ],
        compiler_params=pltpu.CompilerParams(
            dimension_semantics=("parallel",)),
    )(X, center2, csq, xsq)

    X_p = out.reshape(_B, _M, _Dp)
    label = lbl.T.reshape(_B, _M, 1)
    return (X_p, out, center, label, weight)


# speedup vs baseline: 2.4828x; 1.0734x over previous
import functools

import jax
import jax.numpy as jnp
from jax import lax
from jax.experimental import pallas as pl
from jax.experimental.pallas import tpu as pltpu
from jax.experimental.pallas import tpu_sc as plsc

_M = 8
_K = 1024
_D = 1024
_Dp = _D // _M
_B = 4096
_BLK = 512


def _rownorm_s8tree(sq):
    v = sq.reshape(sq.shape[:-1] + (16, 8))
    acc = v[..., 0, :]
    for i in range(1, 16):
        acc = acc + v[..., i, :]
    t1 = acc[..., 0:4] + acc[..., 4:8]
    t2 = t1[..., 0:2] + t1[..., 2:4]
    return t2[..., 0] + t2[..., 1]


def _label_body(x_ref, c2_ref, csq_ref, xsq_ref, lbl_ref):
    labels = []
    for m in range(_M):
        xm = x_ref[:, m * _Dp:(m + 1) * _Dp]
        c2m = c2_ref[m]
        s2 = lax.dot_general(c2m, xm, (((1,), (1,)), ((), ())),
                             preferred_element_type=jnp.float32)
        dist = (csq_ref[m][:, None] - s2) + xsq_ref[m][None, :]
        sq = jnp.sqrt(dist)
        sqb = lax.bitcast_convert_type(sq, jnp.int32)
        colmin = jnp.min(sqb, axis=0, keepdims=True)
        iota = lax.broadcasted_iota(jnp.int32, (_K, _BLK), 0)
        lbl = jnp.min(jnp.where(sqb == colmin, iota, _K), axis=0)
        labels.append(lbl)
    lbl_ref[...] = jnp.stack(labels, axis=0)


_NC = 2
_NW = 32
_ROWS = _B * _M
_BPW = _ROWS // _NW
_CHUNK = 512


def _gather_body(table_hbm, idx_hbm, out_hbm, idx_v, rows_v, sem):
    wid = lax.axis_index("s") * _NC + lax.axis_index("c")
    base = wid * _BPW
    nc = _BPW // _CHUNK
    for c in range(nc):
        pltpu.sync_copy(idx_hbm.at[pl.ds(base + c * _CHUNK, _CHUNK)], idx_v)
        pltpu.async_copy(table_hbm.at[idx_v], rows_v, sem).wait()
        pltpu.sync_copy(rows_v, out_hbm.at[pl.ds(base + c * _CHUNK, _CHUNK)])


_sc_gather = functools.partial(
    pl.kernel,
    out_type=jax.ShapeDtypeStruct((_ROWS, _Dp), jnp.float32),
    mesh=plsc.VectorSubcoreMesh(core_axis_name="c", subcore_axis_name="s"),
    scratch_types=[
        pltpu.VMEM((_CHUNK,), jnp.int32),
        pltpu.VMEM((_CHUNK, _Dp), jnp.float32),
        pltpu.SemaphoreType.DMA,
    ],
)(_gather_body)


def kernel(X, m, center, weight):
    Xs = lax.stop_gradient(X)
    xsq = _rownorm_s8tree((Xs * Xs).reshape(_B, _M, _Dp)).T
    csq = _rownorm_s8tree(center * center)
    center2 = center + center

    grid = (_B // _BLK,)
    lbl = pl.pallas_call(
        _label_body,
        grid=grid,
        in_specs=[
            pl.BlockSpec((_BLK, _D), lambda i: (i, 0)),
            pl.BlockSpec((_M, _K, _Dp), lambda i: (0, 0, 0)),
            pl.BlockSpec((_M, _K), lambda i: (0, 0)),
            pl.BlockSpec((_M, _BLK), lambda i: (0, i)),
        ],
        out_specs=pl.BlockSpec((_M, _BLK), lambda i: (0, i)),
        out_shape=jax.ShapeDtypeStruct((_M, _B), jnp.int32),
        compiler_params=pltpu.CompilerParams(
            dimension_semantics=("parallel",)),
    )(X, center2, csq, xsq)

    lblT = lbl.T
    flat_idx = (lblT + (jnp.arange(_M, dtype=jnp.int32) * _K)[None, :]
                ).reshape(_ROWS)

    cu = lax.bitcast_convert_type(center, jnp.uint32)
    r = cu + jnp.uint32(0x7FFF) + ((cu >> 16) & jnp.uint32(1))
    table = lax.bitcast_convert_type(r & jnp.uint32(0xFFFF0000), jnp.float32)
    rows = _sc_gather(table.reshape(_M * _K, _Dp), flat_idx)

    X_p = rows.reshape(_B, _M, _Dp)
    X_p_m = rows.reshape(_B, _D)
    label = lblT.reshape(_B, _M, 1)
    return (X_p, X_p_m, center, label, weight)


# speedup vs baseline: 2.6997x; 1.0874x over previous
import functools

import jax
import jax.numpy as jnp
from jax import lax
from jax.experimental import pallas as pl
from jax.experimental.pallas import tpu as pltpu
from jax.experimental.pallas import tpu_sc as plsc

_M = 8
_K = 1024
_D = 1024
_Dp = _D // _M
_B = 4096
_BLK = 512


def _rownorm_s8tree(sq):
    v = sq.reshape(sq.shape[:-1] + (16, 8))
    acc = v[..., 0, :]
    for i in range(1, 16):
        acc = acc + v[..., i, :]
    t1 = acc[..., 0:4] + acc[..., 4:8]
    t2 = t1[..., 0:2] + t1[..., 2:4]
    return t2[..., 0] + t2[..., 1]


def _label_body(x_ref, c2_ref, csq_ref, xsq_ref, lbl_ref):
    labels = []
    for m in range(_M):
        xm = x_ref[:, m * _Dp:(m + 1) * _Dp]
        c2m = c2_ref[m]
        s2 = lax.dot_general(c2m, xm, (((1,), (1,)), ((), ())),
                             preferred_element_type=jnp.float32)
        dist = (csq_ref[m][:, None] - s2) + xsq_ref[m][None, :]
        sq = jnp.sqrt(dist)
        sqb = lax.bitcast_convert_type(sq, jnp.int32)
        colmin = jnp.min(sqb, axis=0, keepdims=True)
        iota = lax.broadcasted_iota(jnp.int32, (_K, _BLK), 0)
        lbl = jnp.min(jnp.where(sqb == colmin, iota, _K), axis=0)
        labels.append(lbl)
    lbl_ref[...] = jnp.stack(labels, axis=0)


_NC = 2
_NW = 32
_ROWS = _B * _M
_BPW = _ROWS // _NW
_CHUNK = 512


def _gather_body(table_hbm, idx_hbm, out_xp, out_xpm, idx_v, rows_v, sem):
    wid = lax.axis_index("s") * _NC + lax.axis_index("c")
    base = wid * _BPW
    nc = _BPW // _CHUNK
    for c in range(nc):
        start = base + c * _CHUNK
        mc = start // _B
        b0 = start % _B
        pltpu.sync_copy(idx_hbm.at[pl.ds(start, _CHUNK)], idx_v)
        pltpu.async_copy(table_hbm.at[idx_v], rows_v, sem).wait()
        pltpu.sync_copy(rows_v, out_xp.at[pl.ds(b0, _CHUNK), mc])
        pltpu.sync_copy(rows_v, out_xpm.at[pl.ds(b0, _CHUNK),
                                           pl.ds(mc * _Dp, _Dp)])


_sc_gather = functools.partial(
    pl.kernel,
    out_type=[
        jax.ShapeDtypeStruct((_B, _M, _Dp), jnp.float32),
        jax.ShapeDtypeStruct((_B, _D), jnp.float32),
    ],
    mesh=plsc.VectorSubcoreMesh(core_axis_name="c", subcore_axis_name="s"),
    scratch_types=[
        pltpu.VMEM((_CHUNK,), jnp.int32),
        pltpu.VMEM((_CHUNK, _Dp), jnp.float32),
        pltpu.SemaphoreType.DMA,
    ],
)(_gather_body)


def kernel(X, m, center, weight):
    Xs = lax.stop_gradient(X)
    xsq = _rownorm_s8tree((Xs * Xs).reshape(_B, _M, _Dp)).T
    csq = _rownorm_s8tree(center * center)
    center2 = center + center

    grid = (_B // _BLK,)
    lbl = pl.pallas_call(
        _label_body,
        grid=grid,
        in_specs=[
            pl.BlockSpec((_BLK, _D), lambda i: (i, 0)),
            pl.BlockSpec((_M, _K, _Dp), lambda i: (0, 0, 0)),
            pl.BlockSpec((_M, _K), lambda i: (0, 0)),
            pl.BlockSpec((_M, _BLK), lambda i: (0, i)),
        ],
        out_specs=pl.BlockSpec((_M, _BLK), lambda i: (0, i)),
        out_shape=jax.ShapeDtypeStruct((_M, _B), jnp.int32),
        compiler_params=pltpu.CompilerParams(
            dimension_semantics=("parallel",)),
    )(X, center2, csq, xsq)

    flat_idx = (lbl + (jnp.arange(_M, dtype=jnp.int32) * _K)[:, None]
                ).reshape(_ROWS)

    cu = lax.bitcast_convert_type(center, jnp.uint32)
    r = cu + jnp.uint32(0x7FFF) + ((cu >> 16) & jnp.uint32(1))
    table = lax.bitcast_convert_type(r & jnp.uint32(0xFFFF0000), jnp.float32)
    X_p, X_p_m = _sc_gather(table.reshape(_M * _K, _Dp), flat_idx)

    label = lbl.T.reshape(_B, _M, 1)
    return (X_p, X_p_m, center, label, weight)
